# baseline (speedup 1.0000x reference)
import jax
import jax.numpy as jnp
from jax.experimental import pallas as pl
from jax.experimental.pallas import tpu as pltpu

_SLOPE = 0.1
_VMEM_LIMIT = 56 * 1024 * 1024


def _leaky(y):
    return jnp.where(y > 0, y, _SLOPE * y)


def _dot(a, w):
    return jnp.dot(a, w, preferred_element_type=jnp.float32)


def _bf(v):
    return v.astype(jnp.bfloat16)


def kernel(x, bb0_w, bb0_b, bb1_w, bb1_b, spp_pre_w, spp_pre_b,
           sc_cv1_w, sc_cv1_b, sc_cv3_w, sc_cv3_b, sc_cv2_w, sc_cv2_b,
           sc_cv4_w1, sc_cv4_w2, sc_cv4_b,
           sc_m0_cv1_w, sc_m0_cv1_b, sc_m0_cv2_w, sc_m0_cv2_b,
           sam_w, sam_b,
           cs_cv1_w, cs_cv1_b, cs_cv3_w, cs_cv3_b, cs_cv2_w, cs_cv2_b,
           cs_cv4_w1, cs_cv4_w2, cs_cv4_b,
           cs_m0_cv1_w, cs_m0_cv1_b, cs_m0_cv2_w, cs_m0_cv2_b,
           cs_m1_cv1_w, cs_m1_cv1_b, cs_m1_cv2_w, cs_m1_cv2_b,
           cs_m2_cv1_w, cs_m2_cv1_b, cs_m2_cv2_w, cs_m2_cv2_b,
           head_w, head_b):
    B, _, H, W = x.shape
    H1, W1 = H // 2, W // 2
    H2, W2 = H1 // 2, W1 // 2
    M = H2 * W2
    C1 = bb0_w.shape[-1]
    C = bb1_w.shape[-1]
    Cs = spp_pre_w.shape[-1]
    Ch = head_w.shape[-1]

    K1 = 27
    K1p = 32
    xp8 = jnp.pad(x, ((0, 0), (0, 0), (1, 1), (1, 3)))
    x8 = xp8.reshape(B, 3, H1 + 1, 2, W2 + 1, 4)
    planes_t = []
    for dy in range(3):
        for dx in range(3):
            for c in range(3):
                per_par = []
                for jpar in range(2):
                    o = 2 * jpar + dx
                    per_par.append(x8[:, c, dy // 2:dy // 2 + H1, dy % 2,
                                      o // 4:o // 4 + W2, o % 4])
                planes_t.append(jnp.stack(per_par, axis=2))
    patT = jnp.stack(planes_t, axis=1).astype(jnp.bfloat16)
    patT = jnp.pad(patT, ((0, 0), (0, K1p - K1), (0, 0), (0, 0), (0, 0)))
    pat = patT.reshape(B, K1p, H1 * W1)

    w1p = jnp.pad(bb0_w.reshape(K1, C1).astype(jnp.bfloat16),
                  ((0, K1p - K1), (0, 0)))

    def b_(v):
        return v.astype(jnp.float32).reshape(1, -1)

    weights = [
        w1p, b_(bb0_b),
        bb1_w.astype(jnp.bfloat16), b_(bb1_b),
        spp_pre_w.astype(jnp.bfloat16), b_(spp_pre_b),
        sc_cv1_w.astype(jnp.bfloat16), b_(sc_cv1_b),
        sc_m0_cv1_w.astype(jnp.bfloat16), b_(sc_m0_cv1_b),
        sc_m0_cv2_w.astype(jnp.bfloat16), b_(sc_m0_cv2_b),
        sc_cv3_w.astype(jnp.bfloat16), b_(sc_cv3_b),
        sc_cv2_w.astype(jnp.bfloat16), b_(sc_cv2_b),
        sc_cv4_w1.astype(jnp.bfloat16), sc_cv4_w2.astype(jnp.bfloat16),
        b_(sc_cv4_b),
        sam_w.astype(jnp.bfloat16), b_(sam_b),
        cs_cv1_w.astype(jnp.bfloat16), b_(cs_cv1_b),
        cs_m0_cv1_w.astype(jnp.bfloat16), b_(cs_m0_cv1_b),
        cs_m0_cv2_w.astype(jnp.bfloat16), b_(cs_m0_cv2_b),
        cs_m1_cv1_w.astype(jnp.bfloat16), b_(cs_m1_cv1_b),
        cs_m1_cv2_w.astype(jnp.bfloat16), b_(cs_m1_cv2_b),
        cs_m2_cv1_w.astype(jnp.bfloat16), b_(cs_m2_cv1_b),
        cs_m2_cv2_w.astype(jnp.bfloat16), b_(cs_m2_cv2_b),
        cs_cv3_w.astype(jnp.bfloat16), b_(cs_cv3_b),
        cs_cv2_w.astype(jnp.bfloat16), b_(cs_cv2_b),
        cs_cv4_w1.astype(jnp.bfloat16), cs_cv4_w2.astype(jnp.bfloat16),
        b_(cs_cv4_b),
        head_w.astype(jnp.bfloat16), b_(head_b),
    ]

    def body(pat_ref, w1p_r, b1_r, w2_r, b2_r, wsp_r, bsp_r,
             sc1w, sc1b, sm1w, sm1b, sm2w, sm2b, sc3w, sc3b, sc2w, sc2b,
             sc41, sc42, sc4b, samw, samb,
             cc1w, cc1b, cm0a, cm0ab, cm0b, cm0bb, cm1a, cm1ab, cm1b, cm1bb,
             cm2a, cm2ab, cm2b, cm2bb, cc3w, cc3b, cc2w, cc2b,
             cc41, cc42, cc4b, hw, hb, o_ref):
        y1f = jax.lax.dot_general(
            pat_ref[...], w1p_r[...], (((0,), (0,)), ((), ())),
            preferred_element_type=jnp.float32)
        y1 = _bf(_leaky(y1f + b1_r[...]))
        y5 = y1.reshape(H2, 2, 2, W2, C1)
        planes = [[jnp.pad(y5[:, p, q], ((1, 0), (1, 0), (0, 0)))
                   for q in range(2)] for p in range(2)]
        PSEL = (1, 0, 1)
        OFF = (0, 1, 1)
        acc = None
        for dy in range(3):
            for dx in range(3):
                tp = planes[PSEL[dy]][PSEL[dx]]
                t = tp[OFF[dy]:OFF[dy] + H2,
                       OFF[dx]:OFF[dx] + W2, :].reshape(M, C1)
                d = _dot(t, w2_r[dy, dx])
                acc = d if acc is None else acc + d
        xf = _bf(_leaky(acc + b2_r[...]))

        x2 = _bf(_leaky(_dot(xf, wsp_r[...]) + bsp_r[...]))
        x2s = x2.reshape(H2, W2, Cs)
        neg = jnp.asarray(-jnp.inf, jnp.bfloat16)
        xp6 = jnp.pad(x2s, ((6, 6), (6, 6), (0, 0)), constant_values=neg)

        def rowext(base, offs):
            r = base
            for d in offs:
                r = jnp.maximum(r, xp6[6 + d:6 + d + H2, :, :])
            return r

        row5 = rowext(xp6[4:4 + H2, :, :], (-1, 0, 1, 2))
        row9 = rowext(row5, (-4, -3, 3, 4))
        row13 = rowext(row9, (-6, -5, 5, 6))

        def colred(row, half):
            out = row[:, 6 - half:6 - half + W2, :]
            for d in range(-half + 1, half + 1):
                out = jnp.maximum(out, row[:, 6 + d:6 + d + W2, :])
            return out

        p5 = colred(row5, 2).reshape(M, Cs)
        p9 = colred(row9, 4).reshape(M, Cs)
        p13 = colred(row13, 6).reshape(M, Cs)
        xs4 = (x2, p5, p9, p13)

        def msum(xs_, wref, b):
            a = None
            for i, xi in enumerate(xs_):
                d = _dot(xi, wref[i * Cs:(i + 1) * Cs])
                a = d if a is None else a + d
            return a + b

        def conv3s1(t2d, wref, b):
            t = t2d.reshape(H2, W2, Cs)
            tp = jnp.pad(t, ((1, 1), (1, 1), (0, 0)))
            a = None
            for dy in range(3):
                for dx in range(3):
                    s = tp[dy:dy + H2, dx:dx + W2, :].reshape(M, Cs)
                    d = _dot(s, wref[dy, dx])
                    a = d if a is None else a + d
            return _bf(_leaky(a + b))

        y1c = _bf(_leaky(msum(xs4, sc1w, sc1b[...])))
        t = _bf(_leaky(_dot(y1c, sm1w[...]) + sm1b[...]))
        y1c = conv3s1(t, sm2w, sm2b[...])
        a1 = _bf(_leaky(_dot(y1c, sc3w[...]) + sc3b[...]))
        a2 = _bf(_leaky(msum(xs4, sc2w, sc2b[...])))
        xc = _bf(_leaky(_dot(a1, sc41[...]) + _dot(a2, sc42[...])
                        + sc4b[...]))

        g = jax.nn.sigmoid(_dot(xc, samw[...]) + samb[...])
        xc = _bf(g * xc.astype(jnp.float32))

        y1c = _bf(_leaky(_dot(xc, cc1w[...]) + cc1b[...]))
        for wa, ba, wb, bb in ((cm0a, cm0ab, cm0b, cm0bb),
                               (cm1a, cm1ab, cm1b, cm1bb),
                               (cm2a, cm2ab, cm2b, cm2bb)):
            t = _bf(_leaky(_dot(y1c, wa[...]) + ba[...]))
            y1c = conv3s1(t, wb, bb[...])
        a1 = _bf(_leaky(_dot(y1c, cc3w[...]) + cc3b[...]))
        a2 = _bf(_leaky(_dot(xc, cc2w[...]) + cc2b[...]))
        xo = _bf(_leaky(_dot(a1, cc41[...]) + _dot(a2, cc42[...])
                        + cc4b[...]))

        o_ref[...] = _dot(xo, hw[...]) + hb[...]

    in_specs = [pl.BlockSpec((None, K1p, H1 * W1),
                             lambda i: (i, 0, 0))]
    for wgt in weights:
        nd = wgt.ndim
        in_specs.append(
            pl.BlockSpec(wgt.shape, lambda i, _n=nd: (0,) * _n))

    out = pl.pallas_call(
        body,
        out_shape=jax.ShapeDtypeStruct((B, M, Ch), jnp.float32),
        grid_spec=pltpu.PrefetchScalarGridSpec(
            num_scalar_prefetch=0,
            grid=(B,),
            in_specs=in_specs,
            out_specs=pl.BlockSpec((None, M, Ch), lambda i: (i, 0, 0)),
        ),
        compiler_params=pltpu.CompilerParams(
            dimension_semantics=("parallel",),
            vmem_limit_bytes=_VMEM_LIMIT,
        ),
    )(pat, *weights)

    return jnp.transpose(out.reshape(B, H2, W2, Ch), (0, 3, 1, 2))


# speedup vs baseline: 1.2786x; 1.2786x over previous
import jax
import jax.numpy as jnp
from jax.experimental import pallas as pl
from jax.experimental.pallas import tpu as pltpu

_SLOPE = 0.1
_VMEM_LIMIT = 56 * 1024 * 1024


def _leaky(y):
    return jnp.where(y > 0, y, _SLOPE * y)


def _dot(a, w):
    return jnp.dot(a, w, preferred_element_type=jnp.float32)


def _bf(v):
    return v.astype(jnp.bfloat16)


def kernel(x, bb0_w, bb0_b, bb1_w, bb1_b, spp_pre_w, spp_pre_b,
           sc_cv1_w, sc_cv1_b, sc_cv3_w, sc_cv3_b, sc_cv2_w, sc_cv2_b,
           sc_cv4_w1, sc_cv4_w2, sc_cv4_b,
           sc_m0_cv1_w, sc_m0_cv1_b, sc_m0_cv2_w, sc_m0_cv2_b,
           sam_w, sam_b,
           cs_cv1_w, cs_cv1_b, cs_cv3_w, cs_cv3_b, cs_cv2_w, cs_cv2_b,
           cs_cv4_w1, cs_cv4_w2, cs_cv4_b,
           cs_m0_cv1_w, cs_m0_cv1_b, cs_m0_cv2_w, cs_m0_cv2_b,
           cs_m1_cv1_w, cs_m1_cv1_b, cs_m1_cv2_w, cs_m1_cv2_b,
           cs_m2_cv1_w, cs_m2_cv1_b, cs_m2_cv2_w, cs_m2_cv2_b,
           head_w, head_b):
    B, _, H, W = x.shape
    H1, W1 = H // 2, W // 2
    H2, W2 = H1 // 2, W1 // 2
    M = H2 * W2
    C1 = bb0_w.shape[-1]
    C = bb1_w.shape[-1]
    Cs = spp_pre_w.shape[-1]
    Ch = head_w.shape[-1]

    K1 = 27
    K1p = 32

    def _lo(u):
        return jax.lax.bitcast_convert_type(
            u << jnp.uint32(16), jnp.float32).astype(jnp.bfloat16)

    def _hi(u):
        return jax.lax.bitcast_convert_type(
            u & jnp.uint32(0xFFFF0000), jnp.float32).astype(jnp.bfloat16)

    xb = x.astype(jnp.bfloat16)
    xu = jax.lax.bitcast_convert_type(
        xb.reshape(B, 3, H, W // 2, 2), jnp.uint32)
    ce, co = _lo(xu), _hi(xu)
    ceu = jax.lax.bitcast_convert_type(
        ce.reshape(B, 3, H, W // 4, 2), jnp.uint32)
    cou = jax.lax.bitcast_convert_type(
        co.reshape(B, 3, H, W // 4, 2), jnp.uint32)
    phases = [_lo(ceu), _lo(cou), _hi(ceu), _hi(cou)]
    Rm = [p.reshape(B, 3, H1, 2 * W2) for p in phases]

    zrow = jnp.zeros((B, 1, W2), jnp.bfloat16)
    zlane = jnp.zeros((B, H1, 1), jnp.bfloat16)

    planes_t = []
    for dy in range(3):
        rp = (1, 0, 1)[dy]
        for dx in range(3):
            for c in range(3):
                per_par = []
                for jpar in range(2):
                    o1 = 2 * jpar + dx - 1
                    m = o1 % 4 if o1 >= 0 else 3
                    src = Rm[m][:, c]
                    if o1 == -1:
                        piece = jnp.concatenate(
                            [zlane, src[:, :, rp * W2:rp * W2 + W2 - 1]],
                            axis=2)
                    else:
                        piece = src[:, :, rp * W2:(rp + 1) * W2]
                    if dy == 0:
                        piece = jnp.concatenate(
                            [zrow, piece[:, :H1 - 1]], axis=1)
                    per_par.append(piece)
                planes_t.append(jnp.stack(per_par, axis=2))
    patT = jnp.stack(planes_t, axis=1)
    patT = jnp.pad(patT, ((0, 0), (0, K1p - K1), (0, 0), (0, 0), (0, 0)))
    pat = patT.reshape(B, K1p, H1 * W1)

    w1p = jnp.pad(bb0_w.reshape(K1, C1).astype(jnp.bfloat16),
                  ((0, K1p - K1), (0, 0)))

    def b_(v):
        return v.astype(jnp.float32).reshape(1, -1)

    weights = [
        w1p, b_(bb0_b),
        bb1_w.astype(jnp.bfloat16), b_(bb1_b),
        spp_pre_w.astype(jnp.bfloat16), b_(spp_pre_b),
        sc_cv1_w.astype(jnp.bfloat16), b_(sc_cv1_b),
        sc_m0_cv1_w.astype(jnp.bfloat16), b_(sc_m0_cv1_b),
        sc_m0_cv2_w.astype(jnp.bfloat16), b_(sc_m0_cv2_b),
        sc_cv3_w.astype(jnp.bfloat16), b_(sc_cv3_b),
        sc_cv2_w.astype(jnp.bfloat16), b_(sc_cv2_b),
        sc_cv4_w1.astype(jnp.bfloat16), sc_cv4_w2.astype(jnp.bfloat16),
        b_(sc_cv4_b),
        sam_w.astype(jnp.bfloat16), b_(sam_b),
        cs_cv1_w.astype(jnp.bfloat16), b_(cs_cv1_b),
        cs_m0_cv1_w.astype(jnp.bfloat16), b_(cs_m0_cv1_b),
        cs_m0_cv2_w.astype(jnp.bfloat16), b_(cs_m0_cv2_b),
        cs_m1_cv1_w.astype(jnp.bfloat16), b_(cs_m1_cv1_b),
        cs_m1_cv2_w.astype(jnp.bfloat16), b_(cs_m1_cv2_b),
        cs_m2_cv1_w.astype(jnp.bfloat16), b_(cs_m2_cv1_b),
        cs_m2_cv2_w.astype(jnp.bfloat16), b_(cs_m2_cv2_b),
        cs_cv3_w.astype(jnp.bfloat16), b_(cs_cv3_b),
        cs_cv2_w.astype(jnp.bfloat16), b_(cs_cv2_b),
        cs_cv4_w1.astype(jnp.bfloat16), cs_cv4_w2.astype(jnp.bfloat16),
        b_(cs_cv4_b),
        head_w.astype(jnp.bfloat16), b_(head_b),
    ]

    def body(pat_ref, w1p_r, b1_r, w2_r, b2_r, wsp_r, bsp_r,
             sc1w, sc1b, sm1w, sm1b, sm2w, sm2b, sc3w, sc3b, sc2w, sc2b,
             sc41, sc42, sc4b, samw, samb,
             cc1w, cc1b, cm0a, cm0ab, cm0b, cm0bb, cm1a, cm1ab, cm1b, cm1bb,
             cm2a, cm2ab, cm2b, cm2bb, cc3w, cc3b, cc2w, cc2b,
             cc41, cc42, cc4b, hw, hb, o_ref):
        y1f = jax.lax.dot_general(
            pat_ref[...], w1p_r[...], (((0,), (0,)), ((), ())),
            preferred_element_type=jnp.float32)
        y1 = _bf(_leaky(y1f + b1_r[...]))
        y5 = y1.reshape(H2, 2, 2, W2, C1)
        planes = [[jnp.pad(y5[:, p, q], ((1, 0), (1, 0), (0, 0)))
                   for q in range(2)] for p in range(2)]
        PSEL = (1, 0, 1)
        OFF = (0, 1, 1)
        acc = None
        for dy in range(3):
            for dx in range(3):
                tp = planes[PSEL[dy]][PSEL[dx]]
                t = tp[OFF[dy]:OFF[dy] + H2,
                       OFF[dx]:OFF[dx] + W2, :].reshape(M, C1)
                d = _dot(t, w2_r[dy, dx])
                acc = d if acc is None else acc + d
        xf = _bf(_leaky(acc + b2_r[...]))

        x2 = _bf(_leaky(_dot(xf, wsp_r[...]) + bsp_r[...]))
        x2s = x2.reshape(H2, W2, Cs)
        neg = jnp.asarray(-jnp.inf, jnp.bfloat16)
        xp6 = jnp.pad(x2s, ((6, 6), (6, 6), (0, 0)), constant_values=neg)

        def rowext(base, offs):
            r = base
            for d in offs:
                r = jnp.maximum(r, xp6[6 + d:6 + d + H2, :, :])
            return r

        row5 = rowext(xp6[4:4 + H2, :, :], (-1, 0, 1, 2))
        row9 = rowext(row5, (-4, -3, 3, 4))
        row13 = rowext(row9, (-6, -5, 5, 6))

        def colred(row, half):
            out = row[:, 6 - half:6 - half + W2, :]
            for d in range(-half + 1, half + 1):
                out = jnp.maximum(out, row[:, 6 + d:6 + d + W2, :])
            return out

        p5 = colred(row5, 2).reshape(M, Cs)
        p9 = colred(row9, 4).reshape(M, Cs)
        p13 = colred(row13, 6).reshape(M, Cs)
        xs4 = (x2, p5, p9, p13)

        def msum(xs_, wref, b):
            a = None
            for i, xi in enumerate(xs_):
                d = _dot(xi, wref[i * Cs:(i + 1) * Cs])
                a = d if a is None else a + d
            return a + b

        def conv3s1(t2d, wref, b):
            t = t2d.reshape(H2, W2, Cs)
            tp = jnp.pad(t, ((1, 1), (1, 1), (0, 0)))
            a = None
            for dy in range(3):
                for dx in range(3):
                    s = tp[dy:dy + H2, dx:dx + W2, :].reshape(M, Cs)
                    d = _dot(s, wref[dy, dx])
                    a = d if a is None else a + d
            return _bf(_leaky(a + b))

        y1c = _bf(_leaky(msum(xs4, sc1w, sc1b[...])))
        t = _bf(_leaky(_dot(y1c, sm1w[...]) + sm1b[...]))
        y1c = conv3s1(t, sm2w, sm2b[...])
        a1 = _bf(_leaky(_dot(y1c, sc3w[...]) + sc3b[...]))
        a2 = _bf(_leaky(msum(xs4, sc2w, sc2b[...])))
        xc = _bf(_leaky(_dot(a1, sc41[...]) + _dot(a2, sc42[...])
                        + sc4b[...]))

        g = jax.nn.sigmoid(_dot(xc, samw[...]) + samb[...])
        xc = _bf(g * xc.astype(jnp.float32))

        y1c = _bf(_leaky(_dot(xc, cc1w[...]) + cc1b[...]))
        for wa, ba, wb, bb in ((cm0a, cm0ab, cm0b, cm0bb),
                               (cm1a, cm1ab, cm1b, cm1bb),
                               (cm2a, cm2ab, cm2b, cm2bb)):
            t = _bf(_leaky(_dot(y1c, wa[...]) + ba[...]))
            y1c = conv3s1(t, wb, bb[...])
        a1 = _bf(_leaky(_dot(y1c, cc3w[...]) + cc3b[...]))
        a2 = _bf(_leaky(_dot(xc, cc2w[...]) + cc2b[...]))
        xo = _bf(_leaky(_dot(a1, cc41[...]) + _dot(a2, cc42[...])
                        + cc4b[...]))

        o_ref[...] = _dot(xo, hw[...]) + hb[...]

    in_specs = [pl.BlockSpec((None, K1p, H1 * W1),
                             lambda i: (i, 0, 0))]
    for wgt in weights:
        nd = wgt.ndim
        in_specs.append(
            pl.BlockSpec(wgt.shape, lambda i, _n=nd: (0,) * _n))

    out = pl.pallas_call(
        body,
        out_shape=jax.ShapeDtypeStruct((B, M, Ch), jnp.float32),
        grid_spec=pltpu.PrefetchScalarGridSpec(
            num_scalar_prefetch=0,
            grid=(B,),
            in_specs=in_specs,
            out_specs=pl.BlockSpec((None, M, Ch), lambda i: (i, 0, 0)),
        ),
        compiler_params=pltpu.CompilerParams(
            dimension_semantics=("parallel",),
            vmem_limit_bytes=_VMEM_LIMIT,
        ),
    )(pat, *weights)

    return jnp.transpose(out.reshape(B, H2, W2, Ch), (0, 3, 1, 2))


# speedup vs baseline: 1.6694x; 1.3057x over previous
import jax
import jax.numpy as jnp
from jax.experimental import pallas as pl
from jax.experimental.pallas import tpu as pltpu

_SLOPE = 0.1
_VMEM_LIMIT = 56 * 1024 * 1024


def _leaky(y):
    return jnp.maximum(y, _SLOPE * y)


def _dot(a, w):
    return jnp.dot(a, w, preferred_element_type=jnp.float32)


def _bf(v):
    return v.astype(jnp.bfloat16)


def kernel(x, bb0_w, bb0_b, bb1_w, bb1_b, spp_pre_w, spp_pre_b,
           sc_cv1_w, sc_cv1_b, sc_cv3_w, sc_cv3_b, sc_cv2_w, sc_cv2_b,
           sc_cv4_w1, sc_cv4_w2, sc_cv4_b,
           sc_m0_cv1_w, sc_m0_cv1_b, sc_m0_cv2_w, sc_m0_cv2_b,
           sam_w, sam_b,
           cs_cv1_w, cs_cv1_b, cs_cv3_w, cs_cv3_b, cs_cv2_w, cs_cv2_b,
           cs_cv4_w1, cs_cv4_w2, cs_cv4_b,
           cs_m0_cv1_w, cs_m0_cv1_b, cs_m0_cv2_w, cs_m0_cv2_b,
           cs_m1_cv1_w, cs_m1_cv1_b, cs_m1_cv2_w, cs_m1_cv2_b,
           cs_m2_cv1_w, cs_m2_cv1_b, cs_m2_cv2_w, cs_m2_cv2_b,
           head_w, head_b):
    B, _, H, W = x.shape
    H1, W1 = H // 2, W // 2
    H2, W2 = H1 // 2, W1 // 2
    M = H2 * W2
    C1 = bb0_w.shape[-1]
    C = bb1_w.shape[-1]
    Cs = spp_pre_w.shape[-1]
    Ch = head_w.shape[-1]

    K1 = 27
    K1p = 32

    def _lo(u):
        return jax.lax.bitcast_convert_type(
            u << jnp.uint32(16), jnp.float32).astype(jnp.bfloat16)

    def _hi(u):
        return jax.lax.bitcast_convert_type(
            u & jnp.uint32(0xFFFF0000), jnp.float32).astype(jnp.bfloat16)

    xb = x.astype(jnp.bfloat16)
    xu = jax.lax.bitcast_convert_type(
        xb.reshape(B, 3, H, W // 2, 2), jnp.uint32)
    ce, co = _lo(xu), _hi(xu)
    ceu = jax.lax.bitcast_convert_type(
        ce.reshape(B, 3, H, W // 4, 2), jnp.uint32)
    cou = jax.lax.bitcast_convert_type(
        co.reshape(B, 3, H, W // 4, 2), jnp.uint32)
    phases = [_lo(ceu), _lo(cou), _hi(ceu), _hi(cou)]
    Rm = [p.reshape(B, 3, H1, 2 * W2) for p in phases]

    zrow = jnp.zeros((B, 1, W2), jnp.bfloat16)
    zlane = jnp.zeros((B, H1, 1), jnp.bfloat16)

    planes_t = []
    for dy in range(3):
        rp = (1, 0, 1)[dy]
        for dx in range(3):
            for c in range(3):
                per_par = []
                for jpar in range(2):
                    o1 = 2 * jpar + dx - 1
                    m = o1 % 4 if o1 >= 0 else 3
                    src = Rm[m][:, c]
                    if o1 == -1:
                        piece = jnp.concatenate(
                            [zlane, src[:, :, rp * W2:rp * W2 + W2 - 1]],
                            axis=2)
                    else:
                        piece = src[:, :, rp * W2:(rp + 1) * W2]
                    if dy == 0:
                        piece = jnp.concatenate(
                            [zrow, piece[:, :H1 - 1]], axis=1)
                    per_par.append(piece)
                planes_t.append(jnp.stack(per_par, axis=2))
    patT = jnp.stack(planes_t, axis=1)
    patT = jnp.pad(patT, ((0, 0), (0, K1p - K1), (0, 0), (0, 0), (0, 0)))
    pat = patT.reshape(B, K1p, H1 * W1)

    w1p = jnp.pad(bb0_w.reshape(K1, C1).astype(jnp.bfloat16),
                  ((0, K1p - K1), (0, 0)))

    def b_(v):
        return v.astype(jnp.float32).reshape(1, -1)

    def pair9(w):
        Ci, Co = w.shape[2], w.shape[3]
        wf = w.astype(jnp.bfloat16).reshape(9, Ci, Co)
        return wf[:8].reshape(4, 2 * Ci, Co), wf[8]

    w2p, w2l = pair9(bb1_w)
    smp, sml = pair9(sc_m0_cv2_w)
    cm0p, cm0l = pair9(cs_m0_cv2_w)
    cm1p, cm1l = pair9(cs_m1_cv2_w)
    cm2p, cm2l = pair9(cs_m2_cv2_w)
    sc4 = jnp.concatenate([sc_cv4_w1, sc_cv4_w2], 0).astype(jnp.bfloat16)
    cc4 = jnp.concatenate([cs_cv4_w1, cs_cv4_w2], 0).astype(jnp.bfloat16)

    weights = [
        w1p, b_(bb0_b),
        w2p, w2l, b_(bb1_b),
        spp_pre_w.astype(jnp.bfloat16), b_(spp_pre_b),
        sc_cv1_w.astype(jnp.bfloat16), b_(sc_cv1_b),
        sc_m0_cv1_w.astype(jnp.bfloat16), b_(sc_m0_cv1_b),
        smp, sml, b_(sc_m0_cv2_b),
        sc_cv3_w.astype(jnp.bfloat16), b_(sc_cv3_b),
        sc_cv2_w.astype(jnp.bfloat16), b_(sc_cv2_b),
        sc4, b_(sc_cv4_b),
        sam_w.astype(jnp.bfloat16), b_(sam_b),
        cs_cv1_w.astype(jnp.bfloat16), b_(cs_cv1_b),
        cs_m0_cv1_w.astype(jnp.bfloat16), b_(cs_m0_cv1_b),
        cm0p, cm0l, b_(cs_m0_cv2_b),
        cs_m1_cv1_w.astype(jnp.bfloat16), b_(cs_m1_cv1_b),
        cm1p, cm1l, b_(cs_m1_cv2_b),
        cs_m2_cv1_w.astype(jnp.bfloat16), b_(cs_m2_cv1_b),
        cm2p, cm2l, b_(cs_m2_cv2_b),
        cs_cv3_w.astype(jnp.bfloat16), b_(cs_cv3_b),
        cs_cv2_w.astype(jnp.bfloat16), b_(cs_cv2_b),
        cc4, b_(cs_cv4_b),
        head_w.astype(jnp.bfloat16), b_(head_b),
    ]

    def body(pat_ref, w1p_r, b1_r, w2p_r, w2l_r, b2_r, wsp_r, bsp_r,
             sc1w, sc1b, sm1w, sm1b, sm2p, sm2l, sm2b, sc3w, sc3b,
             sc2w, sc2b, sc4w, sc4b, samw, samb,
             cc1w, cc1b, cm0a, cm0ab, cm0p, cm0lw, cm0bb,
             cm1a, cm1ab, cm1p, cm1lw, cm1bb,
             cm2a, cm2ab, cm2p, cm2lw, cm2bb, cc3w, cc3b, cc2w, cc2b,
             cc4w, cc4b, hw, hb, o_ref):
        y1f = jax.lax.dot_general(
            pat_ref[...], w1p_r[...], (((0,), (0,)), ((), ())),
            preferred_element_type=jnp.float32)
        y1 = _bf(_leaky(y1f + b1_r[...]))
        y5 = y1.reshape(H2, 2, 2, W2, C1)
        planes = [[jnp.pad(y5[:, p, q], ((1, 0), (1, 0), (0, 0)))
                   for q in range(2)] for p in range(2)]
        PSEL = (1, 0, 1)
        OFF = (0, 1, 1)
        taps2 = []
        for dy in range(3):
            for dx in range(3):
                tp = planes[PSEL[dy]][PSEL[dx]]
                taps2.append(tp[OFF[dy]:OFF[dy] + H2,
                                OFF[dx]:OFF[dx] + W2, :].reshape(M, C1))
        acc = None
        for k in range(4):
            t2 = jnp.concatenate([taps2[2 * k], taps2[2 * k + 1]], axis=-1)
            d = _dot(t2, w2p_r[k])
            acc = d if acc is None else acc + d
        acc = acc + _dot(taps2[8], w2l_r[...])
        xf = _bf(_leaky(acc + b2_r[...]))

        x2 = _bf(_leaky(_dot(xf, wsp_r[...]) + bsp_r[...]))
        x2s = x2.reshape(H2, W2, Cs)
        neg = jnp.asarray(-jnp.inf, jnp.bfloat16)
        xp6 = jnp.pad(x2s, ((6, 6), (6, 6), (0, 0)), constant_values=neg)

        def rowext(base, offs):
            r = base
            for d in offs:
                r = jnp.maximum(r, xp6[6 + d:6 + d + H2, :, :])
            return r

        row5 = rowext(xp6[4:4 + H2, :, :], (-1, 0, 1, 2))
        row9 = rowext(row5, (-4, -3, 3, 4))
        row13 = rowext(row9, (-6, -5, 5, 6))

        def colred(row, half):
            out = row[:, 6 - half:6 - half + W2, :]
            for d in range(-half + 1, half + 1):
                out = jnp.maximum(out, row[:, 6 + d:6 + d + W2, :])
            return out

        p5 = colred(row5, 2).reshape(M, Cs)
        p9 = colred(row9, 4).reshape(M, Cs)
        p13 = colred(row13, 6).reshape(M, Cs)
        xs4 = (x2, p5, p9, p13)

        cat4 = jnp.concatenate(xs4, axis=-1)

        def conv3s1(t2d, wp, wl, b):
            t = t2d.reshape(H2, W2, Cs)
            tp = jnp.pad(t, ((1, 1), (1, 1), (0, 0)))
            s9 = []
            for dy in range(3):
                for dx in range(3):
                    s9.append(tp[dy:dy + H2, dx:dx + W2, :].reshape(M, Cs))
            a = None
            for k in range(4):
                s2 = jnp.concatenate([s9[2 * k], s9[2 * k + 1]], axis=-1)
                d = _dot(s2, wp[k])
                a = d if a is None else a + d
            a = a + _dot(s9[8], wl[...])
            return _bf(_leaky(a + b))

        y1c = _bf(_leaky(_dot(cat4, sc1w[...]) + sc1b[...]))
        t = _bf(_leaky(_dot(y1c, sm1w[...]) + sm1b[...]))
        y1c = conv3s1(t, sm2p, sm2l, sm2b[...])
        a1 = _bf(_leaky(_dot(y1c, sc3w[...]) + sc3b[...]))
        a2 = _bf(_leaky(_dot(cat4, sc2w[...]) + sc2b[...]))
        a12 = jnp.concatenate([a1, a2], axis=-1)
        xc = _bf(_leaky(_dot(a12, sc4w[...]) + sc4b[...]))

        g = jax.nn.sigmoid(_dot(xc, samw[...]) + samb[...])
        xc = _bf(g * xc.astype(jnp.float32))

        y1c = _bf(_leaky(_dot(xc, cc1w[...]) + cc1b[...]))
        for wa, ba, wp_, wl_, bb in ((cm0a, cm0ab, cm0p, cm0lw, cm0bb),
                                     (cm1a, cm1ab, cm1p, cm1lw, cm1bb),
                                     (cm2a, cm2ab, cm2p, cm2lw, cm2bb)):
            t = _bf(_leaky(_dot(y1c, wa[...]) + ba[...]))
            y1c = conv3s1(t, wp_, wl_, bb[...])
        a1 = _bf(_leaky(_dot(y1c, cc3w[...]) + cc3b[...]))
        a2 = _bf(_leaky(_dot(xc, cc2w[...]) + cc2b[...]))
        a12b = jnp.concatenate([a1, a2], axis=-1)
        xo = _bf(_leaky(_dot(a12b, cc4w[...]) + cc4b[...]))

        o_ref[...] = _dot(xo, hw[...]) + hb[...]

    in_specs = [pl.BlockSpec((None, K1p, H1 * W1),
                             lambda i: (i, 0, 0))]
    for wgt in weights:
        nd = wgt.ndim
        in_specs.append(
            pl.BlockSpec(wgt.shape, lambda i, _n=nd: (0,) * _n))

    out = pl.pallas_call(
        body,
        out_shape=jax.ShapeDtypeStruct((B, M, Ch), jnp.float32),
        grid_spec=pltpu.PrefetchScalarGridSpec(
            num_scalar_prefetch=0,
            grid=(B,),
            in_specs=in_specs,
            out_specs=pl.BlockSpec((None, M, Ch), lambda i: (i, 0, 0)),
        ),
        compiler_params=pltpu.CompilerParams(
            dimension_semantics=("parallel",),
            vmem_limit_bytes=_VMEM_LIMIT,
        ),
    )(pat, *weights)

    return jnp.transpose(out.reshape(B, H2, W2, Ch), (0, 3, 1, 2))


# speedup vs baseline: 1.7619x; 1.0554x over previous
import jax
import jax.numpy as jnp
from jax.experimental import pallas as pl
from jax.experimental.pallas import tpu as pltpu

_SLOPE = 0.1
_VMEM_LIMIT = 56 * 1024 * 1024


def _leaky(y):
    return jnp.maximum(y, _SLOPE * y)


def _dot(a, w):
    return jnp.dot(a, w, preferred_element_type=jnp.float32)


def _bf(v):
    return v.astype(jnp.bfloat16)


def kernel(x, bb0_w, bb0_b, bb1_w, bb1_b, spp_pre_w, spp_pre_b,
           sc_cv1_w, sc_cv1_b, sc_cv3_w, sc_cv3_b, sc_cv2_w, sc_cv2_b,
           sc_cv4_w1, sc_cv4_w2, sc_cv4_b,
           sc_m0_cv1_w, sc_m0_cv1_b, sc_m0_cv2_w, sc_m0_cv2_b,
           sam_w, sam_b,
           cs_cv1_w, cs_cv1_b, cs_cv3_w, cs_cv3_b, cs_cv2_w, cs_cv2_b,
           cs_cv4_w1, cs_cv4_w2, cs_cv4_b,
           cs_m0_cv1_w, cs_m0_cv1_b, cs_m0_cv2_w, cs_m0_cv2_b,
           cs_m1_cv1_w, cs_m1_cv1_b, cs_m1_cv2_w, cs_m1_cv2_b,
           cs_m2_cv1_w, cs_m2_cv1_b, cs_m2_cv2_w, cs_m2_cv2_b,
           head_w, head_b):
    B, _, H, W = x.shape
    H1, W1 = H // 2, W // 2
    H2, W2 = H1 // 2, W1 // 2
    M = H2 * W2
    C1 = bb0_w.shape[-1]
    C = bb1_w.shape[-1]
    Cs = spp_pre_w.shape[-1]
    Ch = head_w.shape[-1]

    K1 = 27
    K1p = 32

    def _lo(u):
        return jax.lax.bitcast_convert_type(
            u << jnp.uint32(16), jnp.float32).astype(jnp.bfloat16)

    def _hi(u):
        return jax.lax.bitcast_convert_type(
            u & jnp.uint32(0xFFFF0000), jnp.float32).astype(jnp.bfloat16)

    xb = x.astype(jnp.bfloat16)
    xu = jax.lax.bitcast_convert_type(
        xb.reshape(B, 3, H, W // 2, 2), jnp.uint32)
    ce, co = _lo(xu), _hi(xu)
    ceu = jax.lax.bitcast_convert_type(
        ce.reshape(B, 3, H, W // 4, 2), jnp.uint32)
    cou = jax.lax.bitcast_convert_type(
        co.reshape(B, 3, H, W // 4, 2), jnp.uint32)
    phases = [_lo(ceu), _lo(cou), _hi(ceu), _hi(cou)]
    Rm = [p.reshape(B, 3, H1, 2 * W2) for p in phases]

    zrow = jnp.zeros((B, 1, W2), jnp.bfloat16)
    zlane = jnp.zeros((B, H1, 1), jnp.bfloat16)

    planes_t = []
    for dy in range(3):
        rp = (1, 0, 1)[dy]
        for dx in range(3):
            for c in range(3):
                per_par = []
                for jpar in range(2):
                    o1 = 2 * jpar + dx - 1
                    m = o1 % 4 if o1 >= 0 else 3
                    src = Rm[m][:, c]
                    if o1 == -1:
                        piece = jnp.concatenate(
                            [zlane, src[:, :, rp * W2:rp * W2 + W2 - 1]],
                            axis=2)
                    else:
                        piece = src[:, :, rp * W2:(rp + 1) * W2]
                    if dy == 0:
                        piece = jnp.concatenate(
                            [zrow, piece[:, :H1 - 1]], axis=1)
                    per_par.append(piece)
                planes_t.append(jnp.stack(per_par, axis=2))
    patT = jnp.stack(planes_t, axis=1)
    patT = jnp.pad(patT, ((0, 0), (0, K1p - K1), (0, 0), (0, 0), (0, 0)))
    pat = patT.reshape(B, K1p, H1 * W1)

    w1p = jnp.pad(bb0_w.reshape(K1, C1).astype(jnp.bfloat16),
                  ((0, K1p - K1), (0, 0)))

    def b_(v):
        return v.astype(jnp.float32).reshape(1, -1)

    def pair9(w):
        Ci, Co = w.shape[2], w.shape[3]
        wf = w.astype(jnp.bfloat16).reshape(9, Ci, Co)
        return wf[:8].reshape(4, 2 * Ci, Co), wf[8]

    w2p, w2l = pair9(bb1_w)
    smp, sml = pair9(sc_m0_cv2_w)
    cm0p, cm0l = pair9(cs_m0_cv2_w)
    cm1p, cm1l = pair9(cs_m1_cv2_w)
    cm2p, cm2l = pair9(cs_m2_cv2_w)
    sc4 = jnp.concatenate([sc_cv4_w1, sc_cv4_w2], 0).astype(jnp.bfloat16)
    cc4 = jnp.concatenate([cs_cv4_w1, cs_cv4_w2], 0).astype(jnp.bfloat16)

    weights = [
        w1p, b_(bb0_b),
        w2p, w2l, b_(bb1_b),
        spp_pre_w.astype(jnp.bfloat16), b_(spp_pre_b),
        sc_cv1_w.astype(jnp.bfloat16), b_(sc_cv1_b),
        sc_m0_cv1_w.astype(jnp.bfloat16), b_(sc_m0_cv1_b),
        smp, sml, b_(sc_m0_cv2_b),
        sc_cv3_w.astype(jnp.bfloat16), b_(sc_cv3_b),
        sc_cv2_w.astype(jnp.bfloat16), b_(sc_cv2_b),
        sc4, b_(sc_cv4_b),
        sam_w.astype(jnp.bfloat16), b_(sam_b),
        cs_cv1_w.astype(jnp.bfloat16), b_(cs_cv1_b),
        cs_m0_cv1_w.astype(jnp.bfloat16), b_(cs_m0_cv1_b),
        cm0p, cm0l, b_(cs_m0_cv2_b),
        cs_m1_cv1_w.astype(jnp.bfloat16), b_(cs_m1_cv1_b),
        cm1p, cm1l, b_(cs_m1_cv2_b),
        cs_m2_cv1_w.astype(jnp.bfloat16), b_(cs_m2_cv1_b),
        cm2p, cm2l, b_(cs_m2_cv2_b),
        cs_cv3_w.astype(jnp.bfloat16), b_(cs_cv3_b),
        cs_cv2_w.astype(jnp.bfloat16), b_(cs_cv2_b),
        cc4, b_(cs_cv4_b),
        head_w.astype(jnp.bfloat16), b_(head_b),
    ]

    def body(pat_ref, w1p_r, b1_r, w2p_r, w2l_r, b2_r, wsp_r, bsp_r,
             sc1w, sc1b, sm1w, sm1b, sm2p, sm2l, sm2b, sc3w, sc3b,
             sc2w, sc2b, sc4w, sc4b, samw, samb,
             cc1w, cc1b, cm0a, cm0ab, cm0p, cm0lw, cm0bb,
             cm1a, cm1ab, cm1p, cm1lw, cm1bb,
             cm2a, cm2ab, cm2p, cm2lw, cm2bb, cc3w, cc3b, cc2w, cc2b,
             cc4w, cc4b, hw, hb, o_ref):
        y1f = jax.lax.dot_general(
            pat_ref[...], w1p_r[...], (((0,), (0,)), ((), ())),
            preferred_element_type=jnp.float32)
        y1 = _bf(_leaky(y1f + b1_r[...]))
        y5 = y1.reshape(H2, 2, 2, W2, C1)
        planes = [[jnp.pad(y5[:, p, q], ((1, 0), (1, 0), (0, 0)))
                   for q in range(2)] for p in range(2)]
        PSEL = (1, 0, 1)
        OFF = (0, 1, 1)
        taps2 = []
        for dy in range(3):
            for dx in range(3):
                tp = planes[PSEL[dy]][PSEL[dx]]
                taps2.append(tp[OFF[dy]:OFF[dy] + H2,
                                OFF[dx]:OFF[dx] + W2, :].reshape(M, C1))
        acc = None
        for k in range(4):
            t2 = jnp.concatenate([taps2[2 * k], taps2[2 * k + 1]], axis=-1)
            d = _dot(t2, w2p_r[k])
            acc = d if acc is None else acc + d
        acc = acc + _dot(taps2[8], w2l_r[...])
        xf = _bf(_leaky(acc + b2_r[...]))

        x2 = _bf(_leaky(_dot(xf, wsp_r[...]) + bsp_r[...]))
        x2s = x2.reshape(H2, W2, Cs)
        neg = jnp.asarray(-jnp.inf, jnp.bfloat16)
        xp6 = jnp.pad(x2s, ((6, 6), (6, 6), (0, 0)), constant_values=neg)

        P6 = 12 + H2
        sr2 = jnp.maximum(xp6[0:P6 - 1], xp6[1:P6])
        sr4 = jnp.maximum(sr2[0:P6 - 3], sr2[2:P6 - 1])
        row5 = jnp.maximum(sr4[4:4 + H2], xp6[8:8 + H2])
        row9 = jnp.maximum(row5,
                           jnp.maximum(sr2[2:2 + H2], sr2[9:9 + H2]))
        row13 = jnp.maximum(row9,
                            jnp.maximum(sr2[0:H2], sr2[11:11 + H2]))

        def colred(row, half):
            Q = 12 + W2
            c2 = jnp.maximum(row[:, 0:Q - 1], row[:, 1:Q])
            c4 = jnp.maximum(c2[:, 0:Q - 3], c2[:, 2:Q - 1])
            if half == 2:
                return jnp.maximum(c4[:, 4:4 + W2], row[:, 8:8 + W2])
            c8 = jnp.maximum(c4[:, 0:Q - 7], c4[:, 4:Q - 3])
            if half == 4:
                return jnp.maximum(c8[:, 2:2 + W2], row[:, 10:10 + W2])
            out = jnp.maximum(c8[:, 0:W2], c4[:, 8:8 + W2])
            return jnp.maximum(out, row[:, 12:12 + W2])

        p5 = colred(row5, 2).reshape(M, Cs)
        p9 = colred(row9, 4).reshape(M, Cs)
        p13 = colred(row13, 6).reshape(M, Cs)
        xs4 = (x2, p5, p9, p13)

        cat4 = jnp.concatenate(xs4, axis=-1)

        def conv3s1(t2d, wp, wl, b):
            t = t2d.reshape(H2, W2, Cs)
            tp = jnp.pad(t, ((1, 1), (1, 1), (0, 0)))
            s9 = []
            for dy in range(3):
                for dx in range(3):
                    s9.append(tp[dy:dy + H2, dx:dx + W2, :].reshape(M, Cs))
            a = None
            for k in range(4):
                s2 = jnp.concatenate([s9[2 * k], s9[2 * k + 1]], axis=-1)
                d = _dot(s2, wp[k])
                a = d if a is None else a + d
            a = a + _dot(s9[8], wl[...])
            return _bf(_leaky(a + b))

        y1c = _bf(_leaky(_dot(cat4, sc1w[...]) + sc1b[...]))
        t = _bf(_leaky(_dot(y1c, sm1w[...]) + sm1b[...]))
        y1c = conv3s1(t, sm2p, sm2l, sm2b[...])
        a1 = _bf(_leaky(_dot(y1c, sc3w[...]) + sc3b[...]))
        a2 = _bf(_leaky(_dot(cat4, sc2w[...]) + sc2b[...]))
        a12 = jnp.concatenate([a1, a2], axis=-1)
        xc = _bf(_leaky(_dot(a12, sc4w[...]) + sc4b[...]))

        g = jax.nn.sigmoid(_dot(xc, samw[...]) + samb[...])
        xc = _bf(g * xc.astype(jnp.float32))

        y1c = _bf(_leaky(_dot(xc, cc1w[...]) + cc1b[...]))
        for wa, ba, wp_, wl_, bb in ((cm0a, cm0ab, cm0p, cm0lw, cm0bb),
                                     (cm1a, cm1ab, cm1p, cm1lw, cm1bb),
                                     (cm2a, cm2ab, cm2p, cm2lw, cm2bb)):
            t = _bf(_leaky(_dot(y1c, wa[...]) + ba[...]))
            y1c = conv3s1(t, wp_, wl_, bb[...])
        a1 = _bf(_leaky(_dot(y1c, cc3w[...]) + cc3b[...]))
        a2 = _bf(_leaky(_dot(xc, cc2w[...]) + cc2b[...]))
        a12b = jnp.concatenate([a1, a2], axis=-1)
        xo = _bf(_leaky(_dot(a12b, cc4w[...]) + cc4b[...]))

        o_ref[...] = _dot(xo, hw[...]) + hb[...]

    in_specs = [pl.BlockSpec((None, K1p, H1 * W1),
                             lambda i: (i, 0, 0))]
    for wgt in weights:
        nd = wgt.ndim
        in_specs.append(
            pl.BlockSpec(wgt.shape, lambda i, _n=nd: (0,) * _n))

    out = pl.pallas_call(
        body,
        out_shape=jax.ShapeDtypeStruct((B, M, Ch), jnp.float32),
        grid_spec=pltpu.PrefetchScalarGridSpec(
            num_scalar_prefetch=0,
            grid=(B,),
            in_specs=in_specs,
            out_specs=pl.BlockSpec((None, M, Ch), lambda i: (i, 0, 0)),
        ),
        compiler_params=pltpu.CompilerParams(
            dimension_semantics=("parallel",),
            vmem_limit_bytes=_VMEM_LIMIT,
        ),
    )(pat, *weights)

    return jnp.transpose(out.reshape(B, H2, W2, Ch), (0, 3, 1, 2))


# speedup vs baseline: 1.8234x; 1.0349x over previous
import jax
import jax.numpy as jnp
from jax.experimental import pallas as pl
from jax.experimental.pallas import tpu as pltpu

_SLOPE = 0.1
_VMEM_LIMIT = 56 * 1024 * 1024


def _leaky(y):
    return jnp.maximum(y, _SLOPE * y)


def _dot(a, w):
    return jnp.dot(a, w, preferred_element_type=jnp.float32)


def _bf(v):
    return v.astype(jnp.bfloat16)


def kernel(x, bb0_w, bb0_b, bb1_w, bb1_b, spp_pre_w, spp_pre_b,
           sc_cv1_w, sc_cv1_b, sc_cv3_w, sc_cv3_b, sc_cv2_w, sc_cv2_b,
           sc_cv4_w1, sc_cv4_w2, sc_cv4_b,
           sc_m0_cv1_w, sc_m0_cv1_b, sc_m0_cv2_w, sc_m0_cv2_b,
           sam_w, sam_b,
           cs_cv1_w, cs_cv1_b, cs_cv3_w, cs_cv3_b, cs_cv2_w, cs_cv2_b,
           cs_cv4_w1, cs_cv4_w2, cs_cv4_b,
           cs_m0_cv1_w, cs_m0_cv1_b, cs_m0_cv2_w, cs_m0_cv2_b,
           cs_m1_cv1_w, cs_m1_cv1_b, cs_m1_cv2_w, cs_m1_cv2_b,
           cs_m2_cv1_w, cs_m2_cv1_b, cs_m2_cv2_w, cs_m2_cv2_b,
           head_w, head_b):
    B, _, H, W = x.shape
    H1, W1 = H // 2, W // 2
    H2, W2 = H1 // 2, W1 // 2
    M = H2 * W2
    C1 = bb0_w.shape[-1]
    C = bb1_w.shape[-1]
    Cs = spp_pre_w.shape[-1]
    Ch = head_w.shape[-1]

    K1 = 27
    K1p = 32

    def _lo(u):
        return jax.lax.bitcast_convert_type(
            u << jnp.uint32(16), jnp.float32).astype(jnp.bfloat16)

    def _hi(u):
        return jax.lax.bitcast_convert_type(
            u & jnp.uint32(0xFFFF0000), jnp.float32).astype(jnp.bfloat16)

    xb = x.astype(jnp.bfloat16)
    xu = jax.lax.bitcast_convert_type(
        xb.reshape(B, 3, H, W // 2, 2), jnp.uint32)
    ce, co = _lo(xu), _hi(xu)
    ceu = jax.lax.bitcast_convert_type(
        ce.reshape(B, 3, H, W // 4, 2), jnp.uint32)
    cou = jax.lax.bitcast_convert_type(
        co.reshape(B, 3, H, W // 4, 2), jnp.uint32)
    phases = [_lo(ceu), _lo(cou), _hi(ceu), _hi(cou)]
    Rm = [p.reshape(B, 3, H1, 2 * W2) for p in phases]

    zrow = jnp.zeros((B, 1, W2), jnp.bfloat16)
    zlane = jnp.zeros((B, H1, 1), jnp.bfloat16)

    planes_t = []
    for dy in range(3):
        rp = (1, 0, 1)[dy]
        for dx in range(3):
            for c in range(3):
                per_par = []
                for jpar in range(2):
                    o1 = 2 * jpar + dx - 1
                    m = o1 % 4 if o1 >= 0 else 3
                    src = Rm[m][:, c]
                    if o1 == -1:
                        piece = jnp.concatenate(
                            [zlane, src[:, :, rp * W2:rp * W2 + W2 - 1]],
                            axis=2)
                    else:
                        piece = src[:, :, rp * W2:(rp + 1) * W2]
                    if dy == 0:
                        piece = jnp.concatenate(
                            [zrow, piece[:, :H1 - 1]], axis=1)
                    per_par.append(piece)
                planes_t.append(jnp.concatenate(per_par, axis=2))
    zplane = jnp.zeros((B, H1, 2 * W2), jnp.bfloat16)
    planes_t.extend([zplane] * (K1p - K1))
    patT = jnp.stack(planes_t, axis=1)
    pat = patT.reshape(B, K1p, H1 * W1)

    w1p = jnp.pad(bb0_w.reshape(K1, C1).astype(jnp.bfloat16),
                  ((0, K1p - K1), (0, 0)))

    def b_(v):
        return v.astype(jnp.float32).reshape(1, -1)

    def pair9(w):
        Ci, Co = w.shape[2], w.shape[3]
        wf = w.astype(jnp.bfloat16).reshape(9, Ci, Co)
        return wf[:8].reshape(4, 2 * Ci, Co), wf[8]

    w2p, w2l = pair9(bb1_w)
    smp, sml = pair9(sc_m0_cv2_w)
    cm0p, cm0l = pair9(cs_m0_cv2_w)
    cm1p, cm1l = pair9(cs_m1_cv2_w)
    cm2p, cm2l = pair9(cs_m2_cv2_w)
    sc4 = jnp.concatenate([sc_cv4_w1, sc_cv4_w2], 0).astype(jnp.bfloat16)
    cc4 = jnp.concatenate([cs_cv4_w1, cs_cv4_w2], 0).astype(jnp.bfloat16)

    weights = [
        w1p, b_(bb0_b),
        w2p, w2l, b_(bb1_b),
        spp_pre_w.astype(jnp.bfloat16), b_(spp_pre_b),
        sc_cv1_w.astype(jnp.bfloat16), b_(sc_cv1_b),
        sc_m0_cv1_w.astype(jnp.bfloat16), b_(sc_m0_cv1_b),
        smp, sml, b_(sc_m0_cv2_b),
        sc_cv3_w.astype(jnp.bfloat16), b_(sc_cv3_b),
        sc_cv2_w.astype(jnp.bfloat16), b_(sc_cv2_b),
        sc4, b_(sc_cv4_b),
        sam_w.astype(jnp.bfloat16), b_(sam_b),
        cs_cv1_w.astype(jnp.bfloat16), b_(cs_cv1_b),
        cs_m0_cv1_w.astype(jnp.bfloat16), b_(cs_m0_cv1_b),
        cm0p, cm0l, b_(cs_m0_cv2_b),
        cs_m1_cv1_w.astype(jnp.bfloat16), b_(cs_m1_cv1_b),
        cm1p, cm1l, b_(cs_m1_cv2_b),
        cs_m2_cv1_w.astype(jnp.bfloat16), b_(cs_m2_cv1_b),
        cm2p, cm2l, b_(cs_m2_cv2_b),
        cs_cv3_w.astype(jnp.bfloat16), b_(cs_cv3_b),
        cs_cv2_w.astype(jnp.bfloat16), b_(cs_cv2_b),
        cc4, b_(cs_cv4_b),
        head_w.astype(jnp.bfloat16), b_(head_b),
    ]

    def body(pat_ref, w1p_r, b1_r, w2p_r, w2l_r, b2_r, wsp_r, bsp_r,
             sc1w, sc1b, sm1w, sm1b, sm2p, sm2l, sm2b, sc3w, sc3b,
             sc2w, sc2b, sc4w, sc4b, samw, samb,
             cc1w, cc1b, cm0a, cm0ab, cm0p, cm0lw, cm0bb,
             cm1a, cm1ab, cm1p, cm1lw, cm1bb,
             cm2a, cm2ab, cm2p, cm2lw, cm2bb, cc3w, cc3b, cc2w, cc2b,
             cc4w, cc4b, hw, hb, o_ref):
        y1f = jax.lax.dot_general(
            pat_ref[...], w1p_r[...], (((0,), (0,)), ((), ())),
            preferred_element_type=jnp.float32)
        y1 = _bf(_leaky(y1f + b1_r[...]))
        y5 = y1.reshape(H2, 2, 2, W2, C1)
        planes = [[jnp.pad(y5[:, p, q], ((1, 0), (1, 0), (0, 0)))
                   for q in range(2)] for p in range(2)]
        PSEL = (1, 0, 1)
        OFF = (0, 1, 1)
        taps2 = []
        for dy in range(3):
            for dx in range(3):
                tp = planes[PSEL[dy]][PSEL[dx]]
                taps2.append(tp[OFF[dy]:OFF[dy] + H2,
                                OFF[dx]:OFF[dx] + W2, :].reshape(M, C1))
        acc = None
        for k in range(4):
            t2 = jnp.concatenate([taps2[2 * k], taps2[2 * k + 1]], axis=-1)
            d = _dot(t2, w2p_r[k])
            acc = d if acc is None else acc + d
        acc = acc + _dot(taps2[8], w2l_r[...])
        xf = _bf(_leaky(acc + b2_r[...]))

        x2 = _bf(_leaky(_dot(xf, wsp_r[...]) + bsp_r[...]))
        x2s = x2.reshape(H2, W2, Cs)
        neg = jnp.asarray(-jnp.inf, jnp.bfloat16)
        xp6 = jnp.pad(x2s, ((6, 6), (6, 6), (0, 0)), constant_values=neg)

        P6 = 12 + H2
        sr2 = jnp.maximum(xp6[0:P6 - 1], xp6[1:P6])
        sr4 = jnp.maximum(sr2[0:P6 - 3], sr2[2:P6 - 1])
        row5 = jnp.maximum(sr4[4:4 + H2], xp6[8:8 + H2])
        row9 = jnp.maximum(row5,
                           jnp.maximum(sr2[2:2 + H2], sr2[9:9 + H2]))
        row13 = jnp.maximum(row9,
                            jnp.maximum(sr2[0:H2], sr2[11:11 + H2]))

        def colred(row, half):
            Q = 12 + W2
            c2 = jnp.maximum(row[:, 0:Q - 1], row[:, 1:Q])
            c4 = jnp.maximum(c2[:, 0:Q - 3], c2[:, 2:Q - 1])
            if half == 2:
                return jnp.maximum(c4[:, 4:4 + W2], row[:, 8:8 + W2])
            c8 = jnp.maximum(c4[:, 0:Q - 7], c4[:, 4:Q - 3])
            if half == 4:
                return jnp.maximum(c8[:, 2:2 + W2], row[:, 10:10 + W2])
            out = jnp.maximum(c8[:, 0:W2], c4[:, 8:8 + W2])
            return jnp.maximum(out, row[:, 12:12 + W2])

        p5 = colred(row5, 2).reshape(M, Cs)
        p9 = colred(row9, 4).reshape(M, Cs)
        p13 = colred(row13, 6).reshape(M, Cs)
        xs4 = (x2, p5, p9, p13)

        cat4 = jnp.concatenate(xs4, axis=-1)

        def conv3s1(t2d, wp, wl, b):
            t = t2d.reshape(H2, W2, Cs)
            tp = jnp.pad(t, ((1, 1), (1, 1), (0, 0)))
            s9 = []
            for dy in range(3):
                for dx in range(3):
                    s9.append(tp[dy:dy + H2, dx:dx + W2, :].reshape(M, Cs))
            a = None
            for k in range(4):
                s2 = jnp.concatenate([s9[2 * k], s9[2 * k + 1]], axis=-1)
                d = _dot(s2, wp[k])
                a = d if a is None else a + d
            a = a + _dot(s9[8], wl[...])
            return _bf(_leaky(a + b))

        y1c = _bf(_leaky(_dot(cat4, sc1w[...]) + sc1b[...]))
        t = _bf(_leaky(_dot(y1c, sm1w[...]) + sm1b[...]))
        y1c = conv3s1(t, sm2p, sm2l, sm2b[...])
        a1 = _bf(_leaky(_dot(y1c, sc3w[...]) + sc3b[...]))
        a2 = _bf(_leaky(_dot(cat4, sc2w[...]) + sc2b[...]))
        a12 = jnp.concatenate([a1, a2], axis=-1)
        xc = _bf(_leaky(_dot(a12, sc4w[...]) + sc4b[...]))

        g = jax.nn.sigmoid(_dot(xc, samw[...]) + samb[...])
        xc = _bf(g * xc.astype(jnp.float32))

        y1c = _bf(_leaky(_dot(xc, cc1w[...]) + cc1b[...]))
        for wa, ba, wp_, wl_, bb in ((cm0a, cm0ab, cm0p, cm0lw, cm0bb),
                                     (cm1a, cm1ab, cm1p, cm1lw, cm1bb),
                                     (cm2a, cm2ab, cm2p, cm2lw, cm2bb)):
            t = _bf(_leaky(_dot(y1c, wa[...]) + ba[...]))
            y1c = conv3s1(t, wp_, wl_, bb[...])
        a1 = _bf(_leaky(_dot(y1c, cc3w[...]) + cc3b[...]))
        a2 = _bf(_leaky(_dot(xc, cc2w[...]) + cc2b[...]))
        a12b = jnp.concatenate([a1, a2], axis=-1)
        xo = _bf(_leaky(_dot(a12b, cc4w[...]) + cc4b[...]))

        o_ref[...] = _dot(xo, hw[...]) + hb[...]

    in_specs = [pl.BlockSpec((None, K1p, H1 * W1),
                             lambda i: (i, 0, 0))]
    for wgt in weights:
        nd = wgt.ndim
        in_specs.append(
            pl.BlockSpec(wgt.shape, lambda i, _n=nd: (0,) * _n))

    out = pl.pallas_call(
        body,
        out_shape=jax.ShapeDtypeStruct((B, M, Ch), jnp.float32),
        grid_spec=pltpu.PrefetchScalarGridSpec(
            num_scalar_prefetch=0,
            grid=(B,),
            in_specs=in_specs,
            out_specs=pl.BlockSpec((None, M, Ch), lambda i: (i, 0, 0)),
        ),
        compiler_params=pltpu.CompilerParams(
            dimension_semantics=("parallel",),
            vmem_limit_bytes=_VMEM_LIMIT,
        ),
    )(pat, *weights)

    return jnp.transpose(out.reshape(B, H2, W2, Ch), (0, 3, 1, 2))


# speedup vs baseline: 1.8241x; 1.0004x over previous
import jax
import jax.numpy as jnp
from jax.experimental import pallas as pl
from jax.experimental.pallas import tpu as pltpu

_SLOPE = 0.1
_VMEM_LIMIT = 56 * 1024 * 1024


def _leaky(y):
    return jnp.maximum(y, _SLOPE * y)


def _dot(a, w):
    return jnp.dot(a, w, preferred_element_type=jnp.float32)


def _bf(v):
    return v.astype(jnp.bfloat16)


def kernel(x, bb0_w, bb0_b, bb1_w, bb1_b, spp_pre_w, spp_pre_b,
           sc_cv1_w, sc_cv1_b, sc_cv3_w, sc_cv3_b, sc_cv2_w, sc_cv2_b,
           sc_cv4_w1, sc_cv4_w2, sc_cv4_b,
           sc_m0_cv1_w, sc_m0_cv1_b, sc_m0_cv2_w, sc_m0_cv2_b,
           sam_w, sam_b,
           cs_cv1_w, cs_cv1_b, cs_cv3_w, cs_cv3_b, cs_cv2_w, cs_cv2_b,
           cs_cv4_w1, cs_cv4_w2, cs_cv4_b,
           cs_m0_cv1_w, cs_m0_cv1_b, cs_m0_cv2_w, cs_m0_cv2_b,
           cs_m1_cv1_w, cs_m1_cv1_b, cs_m1_cv2_w, cs_m1_cv2_b,
           cs_m2_cv1_w, cs_m2_cv1_b, cs_m2_cv2_w, cs_m2_cv2_b,
           head_w, head_b):
    B, _, H, W = x.shape
    H1, W1 = H // 2, W // 2
    H2, W2 = H1 // 2, W1 // 2
    M = H2 * W2
    C1 = bb0_w.shape[-1]
    C = bb1_w.shape[-1]
    Cs = spp_pre_w.shape[-1]
    Ch = head_w.shape[-1]

    K1 = 27
    K1p = 32

    def _lo(u):
        return jax.lax.bitcast_convert_type(
            u << jnp.uint32(16), jnp.float32).astype(jnp.bfloat16)

    def _hi(u):
        return jax.lax.bitcast_convert_type(
            u & jnp.uint32(0xFFFF0000), jnp.float32).astype(jnp.bfloat16)

    xb = x.astype(jnp.bfloat16)
    xu = jax.lax.bitcast_convert_type(
        xb.reshape(B, 3, H, W // 2, 2), jnp.uint32)
    ce, co = _lo(xu), _hi(xu)
    ceu = jax.lax.bitcast_convert_type(
        ce.reshape(B, 3, H, W // 4, 2), jnp.uint32)
    cou = jax.lax.bitcast_convert_type(
        co.reshape(B, 3, H, W // 4, 2), jnp.uint32)
    phases = [_lo(ceu), _lo(cou), _hi(ceu), _hi(cou)]
    Rm = [p.reshape(B, 3, H1, 2 * W2) for p in phases]

    zr = jnp.zeros((B, 3, 1, 2 * W2), jnp.bfloat16)
    zc = jnp.zeros((B, 3, H1, 1), jnp.bfloat16)
    Rdn = [jnp.concatenate([zr, r[:, :, :H1 - 1]], axis=2) for r in Rm]

    def colsh(a):
        return jnp.concatenate(
            [zc, a[..., 0:W2 - 1], zc, a[..., W2:2 * W2 - 1]], axis=-1)

    R3s, R3sdn = colsh(Rm[3]), colsh(Rdn[3])

    planes_t = []
    for dy in range(3):
        rp = (1, 0, 1)[dy]
        for dx in range(3):
            for c in range(3):
                per_par = []
                for jpar in range(2):
                    o1 = 2 * jpar + dx - 1
                    if o1 == -1:
                        base = R3sdn if dy == 0 else R3s
                    else:
                        base = Rdn[o1] if dy == 0 else Rm[o1]
                    per_par.append(base[:, c, :, rp * W2:(rp + 1) * W2])
                planes_t.append(jnp.concatenate(per_par, axis=2))
    zplane = jnp.zeros((B, H1, 2 * W2), jnp.bfloat16)
    planes_t.extend([zplane] * (K1p - K1))
    patT = jnp.stack(planes_t, axis=1)
    pat = patT.reshape(B, K1p, H1 * W1)

    w1p = jnp.pad(bb0_w.reshape(K1, C1).astype(jnp.bfloat16),
                  ((0, K1p - K1), (0, 0)))

    def b_(v):
        return v.astype(jnp.float32).reshape(1, -1)

    def pair9(w):
        Ci, Co = w.shape[2], w.shape[3]
        wf = w.astype(jnp.bfloat16).reshape(9, Ci, Co)
        return wf[:8].reshape(4, 2 * Ci, Co), wf[8]

    w2p, w2l = pair9(bb1_w)
    smp, sml = pair9(sc_m0_cv2_w)
    cm0p, cm0l = pair9(cs_m0_cv2_w)
    cm1p, cm1l = pair9(cs_m1_cv2_w)
    cm2p, cm2l = pair9(cs_m2_cv2_w)
    sc4 = jnp.concatenate([sc_cv4_w1, sc_cv4_w2], 0).astype(jnp.bfloat16)
    cc4 = jnp.concatenate([cs_cv4_w1, cs_cv4_w2], 0).astype(jnp.bfloat16)

    weights = [
        w1p, b_(bb0_b),
        w2p, w2l, b_(bb1_b),
        spp_pre_w.astype(jnp.bfloat16), b_(spp_pre_b),
        sc_cv1_w.astype(jnp.bfloat16), b_(sc_cv1_b),
        sc_m0_cv1_w.astype(jnp.bfloat16), b_(sc_m0_cv1_b),
        smp, sml, b_(sc_m0_cv2_b),
        sc_cv3_w.astype(jnp.bfloat16), b_(sc_cv3_b),
        sc_cv2_w.astype(jnp.bfloat16), b_(sc_cv2_b),
        sc4, b_(sc_cv4_b),
        sam_w.astype(jnp.bfloat16), b_(sam_b),
        cs_cv1_w.astype(jnp.bfloat16), b_(cs_cv1_b),
        cs_m0_cv1_w.astype(jnp.bfloat16), b_(cs_m0_cv1_b),
        cm0p, cm0l, b_(cs_m0_cv2_b),
        cs_m1_cv1_w.astype(jnp.bfloat16), b_(cs_m1_cv1_b),
        cm1p, cm1l, b_(cs_m1_cv2_b),
        cs_m2_cv1_w.astype(jnp.bfloat16), b_(cs_m2_cv1_b),
        cm2p, cm2l, b_(cs_m2_cv2_b),
        cs_cv3_w.astype(jnp.bfloat16), b_(cs_cv3_b),
        cs_cv2_w.astype(jnp.bfloat16), b_(cs_cv2_b),
        cc4, b_(cs_cv4_b),
        head_w.astype(jnp.bfloat16), b_(head_b),
    ]

    def body(pat_ref, w1p_r, b1_r, w2p_r, w2l_r, b2_r, wsp_r, bsp_r,
             sc1w, sc1b, sm1w, sm1b, sm2p, sm2l, sm2b, sc3w, sc3b,
             sc2w, sc2b, sc4w, sc4b, samw, samb,
             cc1w, cc1b, cm0a, cm0ab, cm0p, cm0lw, cm0bb,
             cm1a, cm1ab, cm1p, cm1lw, cm1bb,
             cm2a, cm2ab, cm2p, cm2lw, cm2bb, cc3w, cc3b, cc2w, cc2b,
             cc4w, cc4b, hw, hb, o_ref):
        y1f = jax.lax.dot_general(
            pat_ref[...], w1p_r[...], (((0,), (0,)), ((), ())),
            preferred_element_type=jnp.float32)
        y1 = _bf(_leaky(y1f + b1_r[...]))
        y5 = y1.reshape(H2, 2, 2, W2, C1)
        planes = [[jnp.pad(y5[:, p, q], ((1, 0), (1, 0), (0, 0)))
                   for q in range(2)] for p in range(2)]
        PSEL = (1, 0, 1)
        OFF = (0, 1, 1)
        taps2 = []
        for dy in range(3):
            for dx in range(3):
                tp = planes[PSEL[dy]][PSEL[dx]]
                taps2.append(tp[OFF[dy]:OFF[dy] + H2,
                                OFF[dx]:OFF[dx] + W2, :].reshape(M, C1))
        acc = None
        for k in range(4):
            t2 = jnp.concatenate([taps2[2 * k], taps2[2 * k + 1]], axis=-1)
            d = _dot(t2, w2p_r[k])
            acc = d if acc is None else acc + d
        acc = acc + _dot(taps2[8], w2l_r[...])
        xf = _bf(_leaky(acc + b2_r[...]))

        x2 = _bf(_leaky(_dot(xf, wsp_r[...]) + bsp_r[...]))
        x2s = x2.reshape(H2, W2, Cs)
        neg = jnp.asarray(-jnp.inf, jnp.bfloat16)
        xp6 = jnp.pad(x2s, ((6, 6), (6, 6), (0, 0)), constant_values=neg)

        P6 = 12 + H2
        sr2 = jnp.maximum(xp6[0:P6 - 1], xp6[1:P6])
        sr4 = jnp.maximum(sr2[0:P6 - 3], sr2[2:P6 - 1])
        row5 = jnp.maximum(sr4[4:4 + H2], xp6[8:8 + H2])
        row9 = jnp.maximum(row5,
                           jnp.maximum(sr2[2:2 + H2], sr2[9:9 + H2]))
        row13 = jnp.maximum(row9,
                            jnp.maximum(sr2[0:H2], sr2[11:11 + H2]))

        def colred(row, half):
            Q = 12 + W2
            c2 = jnp.maximum(row[:, 0:Q - 1], row[:, 1:Q])
            c4 = jnp.maximum(c2[:, 0:Q - 3], c2[:, 2:Q - 1])
            if half == 2:
                return jnp.maximum(c4[:, 4:4 + W2], row[:, 8:8 + W2])
            c8 = jnp.maximum(c4[:, 0:Q - 7], c4[:, 4:Q - 3])
            if half == 4:
                return jnp.maximum(c8[:, 2:2 + W2], row[:, 10:10 + W2])
            out = jnp.maximum(c8[:, 0:W2], c4[:, 8:8 + W2])
            return jnp.maximum(out, row[:, 12:12 + W2])

        p5 = colred(row5, 2).reshape(M, Cs)
        p9 = colred(row9, 4).reshape(M, Cs)
        p13 = colred(row13, 6).reshape(M, Cs)
        xs4 = (x2, p5, p9, p13)

        cat4 = jnp.concatenate(xs4, axis=-1)

        def conv3s1(t2d, wp, wl, b):
            t = t2d.reshape(H2, W2, Cs)
            tp = jnp.pad(t, ((1, 1), (1, 1), (0, 0)))
            s9 = []
            for dy in range(3):
                for dx in range(3):
                    s9.append(tp[dy:dy + H2, dx:dx + W2, :].reshape(M, Cs))
            a = None
            for k in range(4):
                s2 = jnp.concatenate([s9[2 * k], s9[2 * k + 1]], axis=-1)
                d = _dot(s2, wp[k])
                a = d if a is None else a + d
            a = a + _dot(s9[8], wl[...])
            return _bf(_leaky(a + b))

        y1c = _bf(_leaky(_dot(cat4, sc1w[...]) + sc1b[...]))
        t = _bf(_leaky(_dot(y1c, sm1w[...]) + sm1b[...]))
        y1c = conv3s1(t, sm2p, sm2l, sm2b[...])
        a1 = _bf(_leaky(_dot(y1c, sc3w[...]) + sc3b[...]))
        a2 = _bf(_leaky(_dot(cat4, sc2w[...]) + sc2b[...]))
        a12 = jnp.concatenate([a1, a2], axis=-1)
        xc = _bf(_leaky(_dot(a12, sc4w[...]) + sc4b[...]))

        g = jax.nn.sigmoid(_dot(xc, samw[...]) + samb[...])
        xc = _bf(g * xc.astype(jnp.float32))

        y1c = _bf(_leaky(_dot(xc, cc1w[...]) + cc1b[...]))
        for wa, ba, wp_, wl_, bb in ((cm0a, cm0ab, cm0p, cm0lw, cm0bb),
                                     (cm1a, cm1ab, cm1p, cm1lw, cm1bb),
                                     (cm2a, cm2ab, cm2p, cm2lw, cm2bb)):
            t = _bf(_leaky(_dot(y1c, wa[...]) + ba[...]))
            y1c = conv3s1(t, wp_, wl_, bb[...])
        a1 = _bf(_leaky(_dot(y1c, cc3w[...]) + cc3b[...]))
        a2 = _bf(_leaky(_dot(xc, cc2w[...]) + cc2b[...]))
        a12b = jnp.concatenate([a1, a2], axis=-1)
        xo = _bf(_leaky(_dot(a12b, cc4w[...]) + cc4b[...]))

        o_ref[...] = _dot(xo, hw[...]) + hb[...]

    in_specs = [pl.BlockSpec((None, K1p, H1 * W1),
                             lambda i: (i, 0, 0))]
    for wgt in weights:
        nd = wgt.ndim
        in_specs.append(
            pl.BlockSpec(wgt.shape, lambda i, _n=nd: (0,) * _n))

    out = pl.pallas_call(
        body,
        out_shape=jax.ShapeDtypeStruct((B, M, Ch), jnp.float32),
        grid_spec=pltpu.PrefetchScalarGridSpec(
            num_scalar_prefetch=0,
            grid=(B,),
            in_specs=in_specs,
            out_specs=pl.BlockSpec((None, M, Ch), lambda i: (i, 0, 0)),
        ),
        compiler_params=pltpu.CompilerParams(
            dimension_semantics=("parallel",),
            vmem_limit_bytes=_VMEM_LIMIT,
        ),
    )(pat, *weights)

    return jnp.transpose(out.reshape(B, H2, W2, Ch), (0, 3, 1, 2))
